# baseline (speedup 1.0000x reference)
import functools

import jax
import jax.numpy as jnp
from jax import lax
from jax.experimental import pallas as pl
from jax.experimental.pallas import tpu as pltpu
from jax.experimental.pallas import tpu_sc as plsc

N_HEADS = 16
N_TOK = 2048
N_SEL = N_TOK // 2
ROW_TILE = 512
N_ROW_TILES = N_TOK // ROW_TILE
N_BATCH = 4
D_MODEL = 768
J_CHUNK = 512
LANES = 128
SUBS = 8


def _reduce_body(s_ref, rowp_ref, colp_ref):
    h = pl.program_id(1)
    x = s_ref[0]

    rp = x[:, 0:LANES]
    for k in range(1, N_TOK // LANES):
        rp = rp + x[:, k * LANES:(k + 1) * LANES]
    cp = x[0:SUBS, :]
    for k in range(1, ROW_TILE // SUBS):
        cp = cp + x[k * SUBS:(k + 1) * SUBS, :]

    @pl.when(h == 0)
    def _():
        rowp_ref[0] = rp
        colp_ref[0] = cp

    @pl.when(h != 0)
    def _():
        rowp_ref[0] = rowp_ref[0] + rp
        colp_ref[0] = colp_ref[0] + cp


def _partial_sums(scores):
    return pl.pallas_call(
        _reduce_body,
        grid=(N_ROW_TILES, N_HEADS),
        in_specs=[
            pl.BlockSpec((1, ROW_TILE, N_TOK), lambda i, h: (h, i, 0)),
        ],
        out_specs=[
            pl.BlockSpec((1, ROW_TILE, LANES), lambda i, h: (i, 0, 0)),
            pl.BlockSpec((1, SUBS, N_TOK), lambda i, h: (i, 0, 0)),
        ],
        out_shape=[
            jax.ShapeDtypeStruct((N_ROW_TILES, ROW_TILE, LANES), jnp.float32),
            jax.ShapeDtypeStruct((N_ROW_TILES, SUBS, N_TOK), jnp.float32),
        ],
        compiler_params=pltpu.CompilerParams(
            dimension_semantics=("parallel", "arbitrary"),
        ),
    )(scores)


def _select_body(rowp_ref, colp_ref, ids_ref):
    inv = 1.0 / (N_HEADS * N_TOK)
    parts = []
    for i in range(N_ROW_TILES):
        parts.append(jnp.sum(rowp_ref[i], axis=1).reshape(1, ROW_TILE))
    s1 = jnp.concatenate(parts, axis=1) * inv
    cp = colp_ref[0]
    for i in range(1, N_ROW_TILES):
        cp = cp + colp_ref[i]
    s2 = (jnp.sum(cp, axis=0) * inv).reshape(1, N_TOK)

    m1 = jnp.sum(s1) * (1.0 / N_TOK)
    m2 = jnp.sum(s2) * (1.0 / N_TOK)
    v1 = jnp.sum((s1 - m1) ** 2)
    v2 = jnp.sum((s2 - m2) ** 2)
    sig_row = jnp.where(v1 > v2, s1, s2)
    sig_col = sig_row.reshape(N_TOK, 1)

    ii = lax.broadcasted_iota(jnp.int32, (J_CHUNK, N_TOK), 1)
    jj0 = lax.broadcasted_iota(jnp.int32, (J_CHUNK, N_TOK), 0)
    rank = jnp.zeros((1, N_TOK), jnp.float32)
    for j0 in range(0, N_TOK, J_CHUNK):
        sj = lax.slice(sig_col, (j0, 0), (j0 + J_CHUNK, 1))
        beats = (sj > sig_row) | ((sj == sig_row) & (jj0 + j0 < ii))
        rank = rank + jnp.sum(beats.astype(jnp.float32), axis=0, keepdims=True)
    sel_row = (rank < float(N_SEL)).astype(jnp.float32)
    sel_col = sel_row.reshape(N_TOK, 1)

    pos = jnp.zeros((1, N_TOK), jnp.float32)
    for j0 in range(0, N_TOK, J_CHUNK):
        sj = lax.slice(sel_col, (j0, 0), (j0 + J_CHUNK, 1))
        before = sj * ((jj0 + j0) < ii).astype(jnp.float32)
        pos = pos + jnp.sum(before, axis=0, keepdims=True)
    pos_col = pos.reshape(N_TOK, 1)

    pp = lax.broadcasted_iota(jnp.int32, (J_CHUNK, N_SEL), 1).astype(jnp.float32)
    ids = jnp.zeros((1, N_SEL), jnp.float32)
    for i0 in range(0, N_TOK, J_CHUNK):
        si = lax.slice(sel_col, (i0, 0), (i0 + J_CHUNK, 1))
        pi = lax.slice(pos_col, (i0, 0), (i0 + J_CHUNK, 1))
        ival = lax.broadcasted_iota(jnp.int32, (J_CHUNK, N_SEL), 0
                                    ).astype(jnp.float32) + i0
        contrib = si * (pi == pp).astype(jnp.float32) * ival
        ids = ids + jnp.sum(contrib, axis=0, keepdims=True)
    ids = ids.astype(jnp.int32)
    for b in range(N_BATCH):
        ids_ref[0, pl.ds(b * N_SEL, N_SEL)] = ids[0] + b * N_TOK


def _select_ids(scores):
    rowp, colp = _partial_sums(scores)
    return pl.pallas_call(
        _select_body,
        out_shape=jax.ShapeDtypeStruct((1, N_BATCH * N_SEL), jnp.int32),
    )(rowp, colp)


_NC, _NS = 2, 16
_B_PER_W = (N_BATCH * N_SEL) // (_NC * _NS)


@functools.cache
def _sc_gather_fn():
    mesh = plsc.VectorSubcoreMesh(core_axis_name="c", subcore_axis_name="s")

    @functools.partial(
        pl.kernel,
        mesh=mesh,
        out_type=jax.ShapeDtypeStruct((N_BATCH * N_SEL, D_MODEL), jnp.float32),
        scratch_types=[
            pltpu.VMEM((_B_PER_W,), jnp.int32),
            pltpu.VMEM((_B_PER_W, D_MODEL), jnp.float32),
            pltpu.SemaphoreType.DMA,
        ],
    )
    def _sc_gather(table_hbm, idx_hbm, out_hbm, idx_v, rows_v, sem):
        wid = lax.axis_index("s") * _NC + lax.axis_index("c")
        base = wid * _B_PER_W
        pltpu.sync_copy(idx_hbm.at[pl.ds(base, _B_PER_W)], idx_v)
        pltpu.async_copy(table_hbm.at[idx_v], rows_v, sem).wait()
        pltpu.sync_copy(rows_v, out_hbm.at[pl.ds(base, _B_PER_W)])

    return _sc_gather


def kernel(tokens, scores):
    ids4 = _select_ids(scores).reshape(N_BATCH * N_SEL)
    table = tokens.reshape(N_BATCH * N_TOK, D_MODEL)
    out = _sc_gather_fn()(table, ids4)
    return out.reshape(N_BATCH, N_SEL, D_MODEL)
